# baseline (speedup 1.0000x reference)
import functools

import jax
import jax.numpy as jnp
from jax import lax
from jax.experimental import pallas as pl
from jax.experimental.pallas import tpu as pltpu
from jax.experimental.pallas import tpu_sc as plsc

KC = 8192
DD = 32
NN = 8192
BETA = 0.25
BN = 256
NB = NN // BN

SC_CORES = 2
SC_SUBCORES = 16
SC_WORKERS = SC_CORES * SC_SUBCORES
BPW = NN // SC_WORKERS


def _dist_body(z_ref, e_ref, idx_ref, loss_ref, perp_ref, counts_scr, acc_scr):
    i = pl.program_id(0)

    @pl.when(i == 0)
    def _():
        counts_scr[...] = jnp.zeros((1, KC), jnp.float32)
        acc_scr[0] = 0.0

    z = z_ref[...]
    e = e_ref[...]
    mm = lax.dot_general(z.astype(jnp.bfloat16), e,
                         (((1,), (1,)), ((), ())),
                         preferred_element_type=jnp.float32)
    z_sq = jnp.sum(z * z, axis=1, keepdims=True)
    ones_row = jnp.ones((1, DD), dtype=jnp.float32)
    e_sq_row = lax.dot_general(ones_row, e * e, (((1,), (1,)), ((), ())),
                               preferred_element_type=jnp.float32,
                               precision=lax.Precision.HIGHEST)
    dists = (z_sq + e_sq_row) - 2.0 * mm
    idx = jnp.argmin(dists, axis=1, keepdims=True).astype(jnp.int32)
    dmin = jnp.min(dists, axis=1, keepdims=True)
    idx_ref[...] = idx

    k_row = lax.broadcasted_iota(jnp.int32, (1, KC), 1)
    oh = (idx == k_row).astype(jnp.float32)
    counts_scr[...] += jnp.sum(oh, axis=0, keepdims=True)
    acc_scr[0] += jnp.sum(dmin)

    @pl.when(i == NB - 1)
    def _():
        loss = 1.25 * (acc_scr[0] / (NN * DD))
        p = counts_scr[...] * (1.0 / NN)
        eps = 1e-10
        ent = jnp.sum((p + eps) * jnp.log(p + eps))
        loss_ref[...] = jnp.reshape(loss, (1, 1))
        perp_ref[...] = jnp.reshape(jnp.exp(-ent), (1, 1))


def _dist_argmin(z_flat, E):
    return pl.pallas_call(
        _dist_body,
        grid=(NB,),
        in_specs=[
            pl.BlockSpec((BN, DD), lambda i: (i, 0)),
            pl.BlockSpec((KC, DD), lambda i: (0, 0)),
        ],
        out_specs=[
            pl.BlockSpec((BN, 1), lambda i: (i, 0)),
            pl.BlockSpec((1, 1), lambda i: (0, 0)),
            pl.BlockSpec((1, 1), lambda i: (0, 0)),
        ],
        out_shape=[
            jax.ShapeDtypeStruct((NN, 1), jnp.int32),
            jax.ShapeDtypeStruct((1, 1), jnp.float32),
            jax.ShapeDtypeStruct((1, 1), jnp.float32),
        ],
        scratch_shapes=[
            pltpu.VMEM((1, KC), jnp.float32),
            pltpu.SMEM((1,), jnp.float32),
        ],
    )(z_flat, E)


def _sc_gather(E128, idx):
    mesh = plsc.VectorSubcoreMesh(core_axis_name="c", subcore_axis_name="s")

    @functools.partial(
        pl.kernel,
        mesh=mesh,
        out_type=jax.ShapeDtypeStruct((NN, DD), jnp.float32),
        scratch_types=[
            pltpu.VMEM((BPW,), jnp.int32),
            pltpu.VMEM((BPW, 128), jnp.float32),
            pltpu.VMEM((BPW, DD), jnp.float32),
            pltpu.SemaphoreType.DMA,
        ],
    )
    def k(e_hbm, idx_hbm, zq_hbm, idx_v, rows_v, zq_v, sem):
        wid = lax.axis_index("s") * SC_CORES + lax.axis_index("c")
        base = wid * BPW
        pltpu.sync_copy(idx_hbm.at[pl.ds(base, BPW)], idx_v)
        pltpu.async_copy(e_hbm.at[idx_v], rows_v, sem).wait()

        def extract(i, _):
            zq_v[i, pl.ds(0, 16)] = rows_v[i, pl.ds(0, 16)]
            zq_v[i, pl.ds(16, 16)] = rows_v[i, pl.ds(16, 16)]
            return 0
        lax.fori_loop(0, BPW, extract, 0)

        pltpu.sync_copy(zq_v, zq_hbm.at[pl.ds(base, BPW)])

    return k(E128, idx)


def kernel(z, E):
    zp = jnp.transpose(z, (0, 2, 3, 1))
    z_flat = zp.reshape(NN, DD)
    idx2, loss2, perp2 = _dist_argmin(z_flat, E)
    E128 = jnp.pad(E, ((0, 0), (0, 128 - DD)))
    z_q = _sc_gather(E128, idx2.reshape(NN))
    z_q = z_q.reshape(zp.shape)
    z_q_st = zp + (z_q - zp)
    z_q_out = jnp.transpose(z_q_st, (0, 3, 1, 2))
    return (loss2[0, 0], z_q_out, perp2[0, 0])
